# baseline (speedup 1.0000x reference)
import functools
import jax
import jax.numpy as jnp
import numpy as np
from jax.experimental import pallas as pl
from jax.experimental.pallas import tpu as pltpu
from jax.sharding import Mesh, PartitionSpec as P
from jax.experimental.shard_map import shard_map

B, T, N_IN, N, C = 8, 8, 17400, 1000, 10
KC = 1160
NK = N_IN // KC
NH = N // 2


def _stream_kernel(w_ref, out_ref, g_ref):
    k = pl.program_id(0)

    @pl.when(k == 0)
    def _init():
        g_ref[...] = jnp.zeros_like(g_ref)

    g_ref[...] += w_ref[0:64, :]

    @pl.when(k == NK - 1)
    def _finish():
        out_ref[...] = g_ref[0:B, 0:C]


def _stream_half(wh):
    out = pl.pallas_call(
        _stream_kernel,
        grid=(NK,),
        in_specs=[pl.BlockSpec((KC, NH), lambda k: (k, 0))],
        out_specs=pl.BlockSpec((B, C), lambda k: (0, 0)),
        out_shape=jax.ShapeDtypeStruct((B, C), jnp.float32),
        scratch_shapes=[pltpu.VMEM((64, NH), jnp.float32)],
    )(wh)
    return out[None]


def kernel(x, W_in, W_rec, fc_w, fc_b):
    mesh = Mesh(np.array(jax.devices()[:2]), ('d',))
    f = shard_map(_stream_half, mesh=mesh, check_rep=False,
                  in_specs=(P(None, 'd'),), out_specs=P('d'))
    both = f(W_in)
    return both[0] + both[1]


# speedup vs baseline: 4.4822x; 4.4822x over previous
import jax
import jax.numpy as jnp
from jax.experimental import pallas as pl
from jax.experimental.pallas import tpu as pltpu

B, T, N_IN, N, C = 8, 8, 17400, 1000, 10
DECAY = 0.9
THR = 1.0

KC = 696
NK = N_IN // KC


def _dot(a, b, dims):
    return jax.lax.dot_general(a, b, (dims, ((), ())),
                               preferred_element_type=jnp.float32)


def _billeh_kernel(x_ref, w_ref, wrec_ref, fcw_ref, fcb_ref,
                   out_ref, g_ref, xn_ref):
    k = pl.program_id(0)

    @pl.when(k == 0)
    def _init():
        xf = x_ref[...]
        mn = jnp.min(xf, axis=1, keepdims=True)
        den = jnp.maximum(jnp.max(xf, axis=1, keepdims=True) - mn, 1e-6)
        for j in range(NK):
            xn_ref[j] = (xf[:, j * KC:(j + 1) * KC] - mn) / den
        g_ref[...] = jnp.zeros_like(g_ref)

    g_ref[...] += _dot(xn_ref[k], w_ref[...], ((1,), (0,)))

    @pl.when(k == NK - 1)
    def _finish():
        i_in = g_ref[...].reshape(B, T, N)
        wrec = wrec_ref[...]
        v = jnp.zeros((B, N), jnp.float32)
        s = jnp.zeros((B, N), jnp.float32)
        acc = jnp.zeros((B, N), jnp.float32)
        for t in range(T):
            cur = i_in[:, t, :] + _dot(s, wrec, ((1,), (0,)))
            v = DECAY * v * (1.0 - s) + cur
            s = (v > THR).astype(jnp.float32)
            acc = acc + s
        rates = acc * (1.0 / T)
        logits = _dot(rates, fcw_ref[...], ((1,), (1,)))
        out_ref[...] = logits + fcb_ref[...]


def kernel(x, W_in, W_rec, fc_w, fc_b):
    xf = x.astype(jnp.float32).reshape(B * T, N_IN)
    out = pl.pallas_call(
        _billeh_kernel,
        grid=(NK,),
        in_specs=[
            pl.BlockSpec((B * T, N_IN), lambda k: (0, 0)),
            pl.BlockSpec((KC, N), lambda k: (k, 0)),
            pl.BlockSpec((N, N), lambda k: (0, 0)),
            pl.BlockSpec((C, N), lambda k: (0, 0)),
            pl.BlockSpec((1, C), lambda k: (0, 0)),
        ],
        out_specs=pl.BlockSpec((B, C), lambda k: (0, 0)),
        out_shape=jax.ShapeDtypeStruct((B, C), jnp.float32),
        scratch_shapes=[
            pltpu.VMEM((B * T, N), jnp.float32),
            pltpu.VMEM((NK, B * T, KC), jnp.float32),
        ],
    )(xf, W_in, W_rec, fc_w, fc_b.reshape(1, C))
    return out


# speedup vs baseline: 4.9047x; 1.0943x over previous
import jax
import jax.numpy as jnp
from jax.experimental import pallas as pl
from jax.experimental.pallas import tpu as pltpu

B, T, N_IN, N, C = 8, 8, 17400, 1000, 10
DECAY = 0.9
THR = 1.0

KC = 3480
NK = N_IN // KC


def _dot(a, b, dims):
    return jax.lax.dot_general(a, b, (dims, ((), ())),
                               preferred_element_type=jnp.float32)


def _billeh_kernel(x_ref, w_ref, wrec_ref, fcw_ref, fcb_ref,
                   out_ref, g_ref, xn_ref):
    k = pl.program_id(0)

    @pl.when(k == 0)
    def _init():
        xf = x_ref[...]
        mn = jnp.min(xf, axis=1, keepdims=True)
        den = jnp.maximum(jnp.max(xf, axis=1, keepdims=True) - mn, 1e-6)
        for j in range(NK):
            xn_ref[j] = (xf[:, j * KC:(j + 1) * KC] - mn) / den
        g_ref[...] = jnp.zeros_like(g_ref)

    g_ref[...] += _dot(xn_ref[k], w_ref[...], ((1,), (0,)))

    @pl.when(k == NK - 1)
    def _finish():
        i_in = g_ref[...].reshape(B, T, N)
        wrec = wrec_ref[...]
        v = jnp.zeros((B, N), jnp.float32)
        s = jnp.zeros((B, N), jnp.float32)
        acc = jnp.zeros((B, N), jnp.float32)
        for t in range(T):
            cur = i_in[:, t, :] + _dot(s, wrec, ((1,), (0,)))
            v = DECAY * v * (1.0 - s) + cur
            s = (v > THR).astype(jnp.float32)
            acc = acc + s
        rates = acc * (1.0 / T)
        logits = _dot(rates, fcw_ref[...], ((1,), (1,)))
        out_ref[...] = logits + fcb_ref[...]


def kernel(x, W_in, W_rec, fc_w, fc_b):
    xf = x.astype(jnp.float32).reshape(B * T, N_IN)
    out = pl.pallas_call(
        _billeh_kernel,
        grid=(NK,),
        in_specs=[
            pl.BlockSpec((B * T, N_IN), lambda k: (0, 0)),
            pl.BlockSpec((KC, N), lambda k: (k, 0)),
            pl.BlockSpec((N, N), lambda k: (0, 0)),
            pl.BlockSpec((C, N), lambda k: (0, 0)),
            pl.BlockSpec((1, C), lambda k: (0, 0)),
        ],
        out_specs=pl.BlockSpec((B, C), lambda k: (0, 0)),
        out_shape=jax.ShapeDtypeStruct((B, C), jnp.float32),
        scratch_shapes=[
            pltpu.VMEM((B * T, N), jnp.float32),
            pltpu.VMEM((NK, B * T, KC), jnp.float32),
        ],
    )(xf, W_in, W_rec, fc_w, fc_b.reshape(1, C))
    return out
